# Initial kernel scaffold; baseline (speedup 1.0000x reference)
import jax
import jax.numpy as jnp
from jax.experimental import pallas as pl


def kernel(x, edge_index, GW1, Gb1, GW2, Gb2, GW3, Gb3, LW1, Lb1, LW2, Lb2, LW3, Lb3):
    raise NotImplementedError("write your pallas kernel here")


# speedup vs baseline: 3.7203x; 3.7203x over previous
import functools

import jax
import jax.numpy as jnp
from jax import lax
from jax.experimental import pallas as pl
from jax.experimental.pallas import tpu as pltpu
from jax.experimental.pallas import tpu_sc as plsc

N = 50000
E = 800000
F = 5
H = 64
K = 2

NC = 2
NS = 16
NW = NC * NS

CH = 128
NCHUNK = E // CH
ROWS_PER_TILE = N // NS
ZROWS = 625

EB = 1600
NB = 2000

_SELU_SCALE = 1.0507009873554805
_SELU_ALPHA = 1.6732632423543772


def _selu(h):
    return _SELU_SCALE * jnp.where(h > 0, h, _SELU_ALPHA * (jnp.exp(h) - 1.0))


def _prep_weights(GW1, Gb1, GW2, Gb2, GW3, Gb3, LW1, Lb1, LW2, Lb2, LW3, Lb3):
    f32 = jnp.float32
    W1 = jnp.concatenate([GW1[0], GW1[1], LW1[0], LW1[1]], axis=1)
    W1d = jnp.zeros((16, 4 * H), f32).at[0:F, :].set(W1[0:F, :])
    W1s = jnp.zeros((16, 4 * H), f32).at[0:F, :].set(W1[F:2 * F, :])
    b1 = jnp.concatenate([Gb1[0], Gb1[1], Lb1[0], Lb1[1]])[None, :]

    W2 = jnp.zeros((4 * H, 4 * H), f32)
    for j, w in enumerate([GW2[0], GW2[1], LW2[0], LW2[1]]):
        W2 = W2.at[j * H:(j + 1) * H, j * H:(j + 1) * H].set(w)
    b2 = jnp.concatenate([Gb2[0], Gb2[1], Lb2[0], Lb2[1]])[None, :]

    W3g = jnp.zeros((4 * H, 16), f32)
    W3g = W3g.at[0:H, 0:F].set(GW3[0][:, 0::2])
    W3g = W3g.at[H:2 * H, 8:8 + F].set(GW3[1][:, 0::2])
    W3l = jnp.zeros((4 * H, 16), f32)
    W3l = W3l.at[2 * H:3 * H, 0:F].set(LW3[0][:, 0::2])
    W3l = W3l.at[3 * H:4 * H, 8:8 + F].set(LW3[1][:, 0::2])
    b3g = jnp.zeros((1, 16), f32)
    b3g = b3g.at[0, 0:F].set(Gb3[0][0::2]).at[0, 8:8 + F].set(Gb3[1][0::2])
    b3l = jnp.zeros((1, 16), f32)
    b3l = b3l.at[0, 0:F].set(Lb3[0][0::2]).at[0, 8:8 + F].set(Lb3[1][0::2])
    return W1d, W1s, b1, W2, b2, W3g, W3l, b3g, b3l


def _gather_body(xpad, src, dst, xs_out, xd_out,
                 idx_s, idx_d, rows_s, rows_d, sem_s, sem_d):
    wid = lax.axis_index("s") * NC + lax.axis_index("c")
    nt = (NCHUNK - wid + NW - 1) // NW

    def body(t, carry):
        base = (wid + t * NW) * CH
        pltpu.sync_copy(src.at[pl.ds(base, CH)], idx_s)
        pltpu.sync_copy(dst.at[pl.ds(base, CH)], idx_d)
        cps = pltpu.async_copy(xpad.at[idx_s], rows_s, sem_s)
        cpd = pltpu.async_copy(xpad.at[idx_d], rows_d, sem_d)
        cps.wait()
        cpd.wait()
        pltpu.sync_copy(rows_s, xs_out.at[pl.ds(base, CH)])
        pltpu.sync_copy(rows_d, xd_out.at[pl.ds(base, CH)])
        return carry

    lax.fori_loop(0, nt, body, 0)


def _sc_gather(xpad, src, dst):
    f32 = jnp.float32
    mesh = plsc.VectorSubcoreMesh(
        core_axis_name="c", subcore_axis_name="s", num_cores=NC, num_subcores=NS)
    k = pl.kernel(
        _gather_body,
        out_type=[
            jax.ShapeDtypeStruct((E, 16), f32),
            jax.ShapeDtypeStruct((E, 16), f32),
        ],
        mesh=mesh,
        scratch_types=[
            pltpu.VMEM((CH,), jnp.int32),
            pltpu.VMEM((CH,), jnp.int32),
            pltpu.VMEM((CH, 16), f32),
            pltpu.VMEM((CH, 16), f32),
            pltpu.SemaphoreType.DMA,
            pltpu.SemaphoreType.DMA,
        ],
        compiler_params=pltpu.CompilerParams(use_tc_tiling_on_sc=False),
    )
    return k(xpad, src, dst)


def _mlp_body(xd_ref, xs_ref, w1d_ref, w1s_ref, b1_ref, w2_ref, b2_ref,
              w3g_ref, w3l_ref, b3g_ref, b3l_ref, grad_ref, msg_ref):
    f32 = jnp.float32
    xd = xd_ref[...]
    xs = xs_ref[...]
    h = jnp.dot(xd, w1d_ref[...], preferred_element_type=f32)
    h = h + jnp.dot(xs, w1s_ref[...], preferred_element_type=f32)
    h = _selu(h + b1_ref[...])
    h = _selu(jnp.dot(h, w2_ref[...], preferred_element_type=f32) + b2_ref[...])
    ag = jnp.dot(h, w3g_ref[...], preferred_element_type=f32) + b3g_ref[...]
    al = jnp.dot(h, w3l_ref[...], preferred_element_type=f32) + b3l_ref[...]
    d16 = xd - xs
    g16 = ag * d16
    grad_ref[...] = jnp.concatenate([g16[:, 0:F], g16[:, 8:8 + F]], axis=1)
    msg_ref[...] = al * d16


def _tc_mlp(xd, xs, weights):
    f32 = jnp.float32
    full = lambda shape: pl.BlockSpec(shape, lambda i: (0, 0))
    return pl.pallas_call(
        _mlp_body,
        grid=(E // EB,),
        in_specs=[
            pl.BlockSpec((EB, 16), lambda i: (i, 0)),
            pl.BlockSpec((EB, 16), lambda i: (i, 0)),
            full((16, 4 * H)),
            full((16, 4 * H)),
            full((1, 4 * H)),
            full((4 * H, 4 * H)),
            full((1, 4 * H)),
            full((4 * H, 16)),
            full((4 * H, 16)),
            full((1, 16)),
            full((1, 16)),
        ],
        out_specs=[
            pl.BlockSpec((EB, 2 * F), lambda i: (i, 0)),
            pl.BlockSpec((EB, 16), lambda i: (i, 0)),
        ],
        out_shape=[
            jax.ShapeDtypeStruct((E, 2 * F), f32),
            jax.ShapeDtypeStruct((E, 16), f32),
        ],
        compiler_params=pltpu.CompilerParams(
            dimension_semantics=("arbitrary",)),
    )(xd, xs, *weights)


def _scatter_body(msg, dst, part_out, idx_v, rows_v, zbuf, acc, sem):
    c = lax.axis_index("c")
    s = lax.axis_index("s")
    wid = s * NC + c

    def zrow(i, carry):
        zbuf[i] = jnp.zeros((16,), jnp.float32)
        return carry

    lax.fori_loop(0, ZROWS, zrow, 0)
    for j in range(ROWS_PER_TILE // ZROWS):
        pltpu.sync_copy(zbuf, acc.at[pl.ds(s * ROWS_PER_TILE + j * ZROWS, ZROWS)])
    plsc.subcore_barrier()

    nt = (NCHUNK - wid + NW - 1) // NW

    def body(t, carry):
        base = (wid + t * NW) * CH
        pltpu.sync_copy(dst.at[pl.ds(base, CH)], idx_v)
        pltpu.sync_copy(msg.at[pl.ds(base, CH)], rows_v)
        pltpu.sync_copy(rows_v, acc.at[idx_v], add=True)
        return carry

    lax.fori_loop(0, nt, body, 0)
    plsc.subcore_barrier()
    pltpu.sync_copy(acc.at[pl.ds(s * ROWS_PER_TILE, ROWS_PER_TILE)],
                    part_out.at[c, pl.ds(s * ROWS_PER_TILE, ROWS_PER_TILE)])


def _sc_scatter(msg, dst):
    f32 = jnp.float32
    mesh = plsc.VectorSubcoreMesh(
        core_axis_name="c", subcore_axis_name="s", num_cores=NC, num_subcores=NS)
    k = pl.kernel(
        _scatter_body,
        out_type=jax.ShapeDtypeStruct((NC, N, 16), f32),
        mesh=mesh,
        scratch_types=[
            pltpu.VMEM((CH,), jnp.int32),
            pltpu.VMEM((CH, 16), f32),
            pltpu.VMEM((ZROWS, 16), f32),
            pltpu.VMEM_SHARED((N, 16), f32),
            pltpu.SemaphoreType.DMA,
        ],
        compiler_params=pltpu.CompilerParams(use_tc_tiling_on_sc=False),
    )
    return k(msg, dst)


def _combine_body(part_ref, out_ref):
    p = part_ref[0] + part_ref[1]
    out_ref[...] = jnp.concatenate([p[:, 0:F], p[:, 8:8 + F]], axis=1)


def _tc_combine(part):
    return pl.pallas_call(
        _combine_body,
        grid=(N // NB,),
        in_specs=[pl.BlockSpec((NC, NB, 16), lambda i: (0, i, 0))],
        out_specs=pl.BlockSpec((NB, 2 * F), lambda i: (i, 0)),
        out_shape=jax.ShapeDtypeStruct((N, 2 * F), jnp.float32),
        compiler_params=pltpu.CompilerParams(
            dimension_semantics=("arbitrary",)),
    )(part)


def kernel(x, edge_index, GW1, Gb1, GW2, Gb2, GW3, Gb3,
           LW1, Lb1, LW2, Lb2, LW3, Lb3):
    f32 = jnp.float32
    xpad = jnp.zeros((N, 16), f32)
    xpad = xpad.at[:, 0:F].set(x).at[:, 8:8 + F].set(x)
    src = edge_index[0]
    dst = edge_index[1]

    xs16, xd16 = _sc_gather(xpad, src, dst)
    weights = _prep_weights(GW1, Gb1, GW2, Gb2, GW3, Gb3,
                            LW1, Lb1, LW2, Lb2, LW3, Lb3)
    grad, msg = _tc_mlp(xd16, xs16, weights)
    part = _sc_scatter(msg, dst)
    lap = _tc_combine(part)
    return jnp.concatenate([grad, lap], axis=0)


# speedup vs baseline: 3.8690x; 1.0400x over previous
import functools

import jax
import jax.numpy as jnp
from jax import lax
from jax.experimental import pallas as pl
from jax.experimental.pallas import tpu as pltpu
from jax.experimental.pallas import tpu_sc as plsc

N = 50000
E = 800000
F = 5
H = 64
K = 2

NC = 2
NS = 16
NW = NC * NS

CH = 128
NCHUNK = E // CH
ROWS_PER_TILE = N // NS
ZROWS = 625

EB = 1600
RB = EB // 8
NB = 2000
XB = 2000

_SELU_SCALE = 1.0507009873554805
_SELU_ALPHA = 1.6732632423543772


def _selu(h):
    return _SELU_SCALE * jnp.where(h > 0, h, _SELU_ALPHA * (jnp.exp(h) - 1.0))


def _prep_weights(GW1, Gb1, GW2, Gb2, GW3, Gb3, LW1, Lb1, LW2, Lb2, LW3, Lb3):
    f32 = jnp.float32
    W1 = jnp.concatenate([GW1[0], GW1[1], LW1[0], LW1[1]], axis=1)
    W1d = jnp.zeros((16, 4 * H), f32).at[0:F, :].set(W1[0:F, :])
    W1s = jnp.zeros((16, 4 * H), f32).at[0:F, :].set(W1[F:2 * F, :])
    b1 = jnp.concatenate([Gb1[0], Gb1[1], Lb1[0], Lb1[1]])[None, :]

    W2 = jnp.zeros((4 * H, 4 * H), f32)
    for j, w in enumerate([GW2[0], GW2[1], LW2[0], LW2[1]]):
        W2 = W2.at[j * H:(j + 1) * H, j * H:(j + 1) * H].set(w)
    b2 = jnp.concatenate([Gb2[0], Gb2[1], Lb2[0], Lb2[1]])[None, :]

    W3g = jnp.zeros((4 * H, 16), f32)
    W3g = W3g.at[0:H, 0:F].set(GW3[0][:, 0::2])
    W3g = W3g.at[H:2 * H, 8:8 + F].set(GW3[1][:, 0::2])
    W3l = jnp.zeros((4 * H, 16), f32)
    W3l = W3l.at[2 * H:3 * H, 0:F].set(LW3[0][:, 0::2])
    W3l = W3l.at[3 * H:4 * H, 8:8 + F].set(LW3[1][:, 0::2])
    b3g = jnp.zeros((1, 16), f32)
    b3g = b3g.at[0, 0:F].set(Gb3[0][0::2]).at[0, 8:8 + F].set(Gb3[1][0::2])
    b3l = jnp.zeros((1, 16), f32)
    b3l = b3l.at[0, 0:F].set(Lb3[0][0::2]).at[0, 8:8 + F].set(Lb3[1][0::2])
    return W1d, W1s, b1, W2, b2, W3g, W3l, b3g, b3l


def _pack_body(x_ref, out_ref):
    x = x_ref[...]
    z = jnp.zeros((XB, 3), jnp.float32)
    out_ref[...] = jnp.concatenate([x, z, x, z], axis=1)


def _tc_pack(x):
    return pl.pallas_call(
        _pack_body,
        grid=(N // XB,),
        in_specs=[pl.BlockSpec((XB, F), lambda i: (i, 0))],
        out_specs=pl.BlockSpec((XB, 16), lambda i: (i, 0)),
        out_shape=jax.ShapeDtypeStruct((N, 16), jnp.float32),
        compiler_params=pltpu.CompilerParams(
            dimension_semantics=("arbitrary",)),
    )(x)


def _gather_body(xpad, src, dst, xs_out, xd_out,
                 idx_s, idx_d, rows_s, rows_d, sem_s, sem_d):
    wid = lax.axis_index("s") * NC + lax.axis_index("c")
    nt = (NCHUNK - wid + NW - 1) // NW

    def body(t, carry):
        base = (wid + t * NW) * CH
        pltpu.sync_copy(src.at[pl.ds(base, CH)], idx_s)
        pltpu.sync_copy(dst.at[pl.ds(base, CH)], idx_d)
        cps = pltpu.async_copy(xpad.at[idx_s], rows_s, sem_s)
        cpd = pltpu.async_copy(xpad.at[idx_d], rows_d, sem_d)
        cps.wait()
        cpd.wait()
        pltpu.sync_copy(rows_s, xs_out.at[pl.ds(base, CH)])
        pltpu.sync_copy(rows_d, xd_out.at[pl.ds(base, CH)])
        return carry

    lax.fori_loop(0, nt, body, 0)


def _sc_gather(xpad, src, dst):
    f32 = jnp.float32
    mesh = plsc.VectorSubcoreMesh(
        core_axis_name="c", subcore_axis_name="s", num_cores=NC, num_subcores=NS)
    k = pl.kernel(
        _gather_body,
        out_type=[
            jax.ShapeDtypeStruct((E, 16), f32),
            jax.ShapeDtypeStruct((E, 16), f32),
        ],
        mesh=mesh,
        scratch_types=[
            pltpu.VMEM((CH,), jnp.int32),
            pltpu.VMEM((CH,), jnp.int32),
            pltpu.VMEM((CH, 16), f32),
            pltpu.VMEM((CH, 16), f32),
            pltpu.SemaphoreType.DMA,
            pltpu.SemaphoreType.DMA,
        ],
        compiler_params=pltpu.CompilerParams(use_tc_tiling_on_sc=False),
    )
    return k(xpad, src, dst)


def _mlp_body(xd_ref, xs_ref, w1d_ref, w1s_ref, b1_ref, w2_ref, b2_ref,
              w3g_ref, w3l_ref, b3g_ref, b3l_ref, grad_ref, msg_ref):
    f32 = jnp.float32
    xdp = xd_ref[...]
    xsp = xs_ref[...]
    xd = jnp.concatenate(
        [xdp[:, 16 * j:16 * (j + 1)] for j in range(8)], axis=0)
    xs = jnp.concatenate(
        [xsp[:, 16 * j:16 * (j + 1)] for j in range(8)], axis=0)
    h = jnp.dot(xd, w1d_ref[...], preferred_element_type=f32)
    h = h + jnp.dot(xs, w1s_ref[...], preferred_element_type=f32)
    h = _selu(h + b1_ref[...])
    h = _selu(jnp.dot(h, w2_ref[...], preferred_element_type=f32) + b2_ref[...])
    ag = jnp.dot(h, w3g_ref[...], preferred_element_type=f32) + b3g_ref[...]
    al = jnp.dot(h, w3l_ref[...], preferred_element_type=f32) + b3l_ref[...]
    d16 = xd - xs
    g16 = ag * d16
    m16 = al * d16
    grad_ref[...] = jnp.concatenate(
        [g16[RB * j:RB * (j + 1), :] for j in range(8)], axis=1)
    msg_ref[...] = jnp.concatenate(
        [m16[RB * j:RB * (j + 1), :] for j in range(8)], axis=1)


def _tc_mlp(xd_pk, xs_pk, weights):
    f32 = jnp.float32
    full = lambda shape: pl.BlockSpec(shape, lambda i: (0, 0))
    return pl.pallas_call(
        _mlp_body,
        grid=(E // EB,),
        in_specs=[
            pl.BlockSpec((RB, 128), lambda i: (i, 0)),
            pl.BlockSpec((RB, 128), lambda i: (i, 0)),
            full((16, 4 * H)),
            full((16, 4 * H)),
            full((1, 4 * H)),
            full((4 * H, 4 * H)),
            full((1, 4 * H)),
            full((4 * H, 16)),
            full((4 * H, 16)),
            full((1, 16)),
            full((1, 16)),
        ],
        out_specs=[
            pl.BlockSpec((RB, 128), lambda i: (i, 0)),
            pl.BlockSpec((RB, 128), lambda i: (i, 0)),
        ],
        out_shape=[
            jax.ShapeDtypeStruct((E // 8, 128), f32),
            jax.ShapeDtypeStruct((E // 8, 128), f32),
        ],
        compiler_params=pltpu.CompilerParams(
            dimension_semantics=("arbitrary",)),
    )(xd_pk, xs_pk, *weights)


def _scatter_body(msg, dst, part_out, idx_v, rows_v, zbuf, acc, sem):
    c = lax.axis_index("c")
    s = lax.axis_index("s")
    wid = s * NC + c

    def zrow(i, carry):
        zbuf[i] = jnp.zeros((16,), jnp.float32)
        return carry

    lax.fori_loop(0, ZROWS, zrow, 0)
    for j in range(ROWS_PER_TILE // ZROWS):
        pltpu.sync_copy(zbuf, acc.at[pl.ds(s * ROWS_PER_TILE + j * ZROWS, ZROWS)])
    plsc.subcore_barrier()

    nt = (NCHUNK - wid + NW - 1) // NW

    def body(t, carry):
        base = (wid + t * NW) * CH
        pltpu.sync_copy(dst.at[pl.ds(base, CH)], idx_v)
        pltpu.sync_copy(msg.at[pl.ds(base, CH)], rows_v)
        pltpu.sync_copy(rows_v, acc.at[idx_v], add=True)
        return carry

    lax.fori_loop(0, nt, body, 0)
    plsc.subcore_barrier()
    pltpu.sync_copy(acc.at[pl.ds(s * ROWS_PER_TILE, ROWS_PER_TILE)],
                    part_out.at[c, pl.ds(s * ROWS_PER_TILE, ROWS_PER_TILE)])


def _sc_scatter(msg, dst):
    f32 = jnp.float32
    mesh = plsc.VectorSubcoreMesh(
        core_axis_name="c", subcore_axis_name="s", num_cores=NC, num_subcores=NS)
    k = pl.kernel(
        _scatter_body,
        out_type=jax.ShapeDtypeStruct((NC, N, 16), f32),
        mesh=mesh,
        scratch_types=[
            pltpu.VMEM((CH,), jnp.int32),
            pltpu.VMEM((CH, 16), f32),
            pltpu.VMEM((ZROWS, 16), f32),
            pltpu.VMEM_SHARED((N, 16), f32),
            pltpu.SemaphoreType.DMA,
        ],
        compiler_params=pltpu.CompilerParams(use_tc_tiling_on_sc=False),
    )
    return k(msg, dst)


def _combine_body(part_ref, out_ref):
    p16 = part_ref[0] + part_ref[1]
    out_ref[...] = jnp.concatenate([p16[:, 0:F], p16[:, 8:8 + F]], axis=1)


def _tc_combine(part):
    return pl.pallas_call(
        _combine_body,
        grid=(N // NB,),
        in_specs=[pl.BlockSpec((NC, NB, 16), lambda i: (0, i, 0))],
        out_specs=pl.BlockSpec((NB, 2 * F), lambda i: (i, 0)),
        out_shape=jax.ShapeDtypeStruct((N, 2 * F), jnp.float32),
        compiler_params=pltpu.CompilerParams(
            dimension_semantics=("arbitrary",)),
    )(part)


def kernel(x, edge_index, GW1, Gb1, GW2, Gb2, GW3, Gb3,
           LW1, Lb1, LW2, Lb2, LW3, Lb3):
    src = edge_index[0]
    dst = edge_index[1]

    xpad = _tc_pack(x)
    xs16, xd16 = _sc_gather(xpad, src, dst)
    xs_pk = jnp.reshape(xs16, (E // 8, 128))
    xd_pk = jnp.reshape(xd16, (E // 8, 128))
    weights = _prep_weights(GW1, Gb1, GW2, Gb2, GW3, Gb3,
                            LW1, Lb1, LW2, Lb2, LW3, Lb3)
    grad_pk, msg_pk = _tc_mlp(xd_pk, xs_pk, weights)
    msg16 = jnp.reshape(msg_pk, (E, 16))
    part = _sc_scatter(msg16, dst)
    lap = _tc_combine(part)
    g16 = jnp.reshape(grad_pk, (E, 16))
    grad = jnp.concatenate([g16[:, 0:F], g16[:, 8:8 + F]], axis=1)
    return jnp.concatenate([grad, lap], axis=0)


# speedup vs baseline: 6.3923x; 1.6522x over previous
import functools

import jax
import jax.numpy as jnp
from jax import lax
from jax.experimental import pallas as pl
from jax.experimental.pallas import tpu as pltpu
from jax.experimental.pallas import tpu_sc as plsc

N = 50000
E = 800000
F = 5
H = 64
K = 2

NC = 2
NS = 16
NW = NC * NS

CH = 128
NCHUNK = E // CH
CPW = 200
ROWS_PER_TILE = N // NS
ZROWS = 625

EB = 1280
RB = EB // 8
NB = 2000
XB = 2000

_SELU_SCALE = 1.0507009873554805
_SELU_ALPHA = 1.6732632423543772

_RUNS = []
for _q in range(10):
    _k0 = CH * _q
    _r, _k, _i = [], _k0, 0
    while _k < _k0 + CH:
        _j, _rr = _k // RB, _k % RB
        _L = min(_k0 + CH, (_j + 1) * RB) - _k
        _r.append((_j, _rr, _i, _L))
        _k += _L
        _i += _L
    _RUNS.append(_r)


def _selu(h):
    return _SELU_SCALE * jnp.where(h > 0, h, _SELU_ALPHA * (jnp.exp(h) - 1.0))


def _prep_weights(GW1, Gb1, GW2, Gb2, GW3, Gb3, LW1, Lb1, LW2, Lb2, LW3, Lb3):
    f32 = jnp.float32
    W1 = jnp.concatenate([GW1[0], GW1[1], LW1[0], LW1[1]], axis=1)
    W1d = jnp.zeros((16, 4 * H), f32).at[0:F, :].set(W1[0:F, :])
    W1s = jnp.zeros((16, 4 * H), f32).at[0:F, :].set(W1[F:2 * F, :])
    b1 = jnp.concatenate([Gb1[0], Gb1[1], Lb1[0], Lb1[1]])[None, :]

    W2 = jnp.zeros((4 * H, 4 * H), f32)
    for j, w in enumerate([GW2[0], GW2[1], LW2[0], LW2[1]]):
        W2 = W2.at[j * H:(j + 1) * H, j * H:(j + 1) * H].set(w)
    b2 = jnp.concatenate([Gb2[0], Gb2[1], Lb2[0], Lb2[1]])[None, :]

    W3g = jnp.zeros((4 * H, 16), f32)
    W3g = W3g.at[0:H, 0:F].set(GW3[0][:, 0::2])
    W3g = W3g.at[H:2 * H, 8:8 + F].set(GW3[1][:, 0::2])
    W3l = jnp.zeros((4 * H, 16), f32)
    W3l = W3l.at[2 * H:3 * H, 0:F].set(LW3[0][:, 0::2])
    W3l = W3l.at[3 * H:4 * H, 8:8 + F].set(LW3[1][:, 0::2])
    b3g = jnp.zeros((1, 16), f32)
    b3g = b3g.at[0, 0:F].set(Gb3[0][0::2]).at[0, 8:8 + F].set(Gb3[1][0::2])
    b3l = jnp.zeros((1, 16), f32)
    b3l = b3l.at[0, 0:F].set(Lb3[0][0::2]).at[0, 8:8 + F].set(Lb3[1][0::2])
    return W1d, W1s, b1, W2, b2, W3g, W3l, b3g, b3l


def _pack_body(x_ref, out_ref):
    x = x_ref[...]
    z = jnp.zeros((XB, 3), jnp.float32)
    out_ref[...] = jnp.concatenate([x, z, x, z], axis=1)


def _tc_pack(x):
    return pl.pallas_call(
        _pack_body,
        grid=(N // XB,),
        in_specs=[pl.BlockSpec((XB, F), lambda i: (i, 0))],
        out_specs=pl.BlockSpec((XB, 16), lambda i: (i, 0)),
        out_shape=jax.ShapeDtypeStruct((N, 16), jnp.float32),
        compiler_params=pltpu.CompilerParams(
            dimension_semantics=("arbitrary",)),
    )(x)


def _gather_body(xpad, src, dst, xs_out, xd_out,
                 idx_s, idx_d, rows_s, rows_d, sem_i, sem_g):
    wid = lax.axis_index("s") * NC + lax.axis_index("c")
    g0 = wid * CPW
    nt = lax.select(wid < NW - 1, CPW // 10, (NCHUNK - (NW - 1) * CPW) // 10)

    def body(u, carry):
        blk = wid * (CPW // 10) + u
        base = blk * EB
        cps = []
        for q in range(10):
            cps.append(pltpu.async_copy(
                src.at[pl.ds(base + CH * q, CH)], idx_s.at[q], sem_i))
            cps.append(pltpu.async_copy(
                dst.at[pl.ds(base + CH * q, CH)], idx_d.at[q], sem_i))
        for cp in cps:
            cp.wait()
        cps = []
        for q in range(10):
            cps.append(pltpu.async_copy(
                xpad.at[idx_s.at[q]], rows_s.at[pl.ds(CH * q, CH)], sem_g))
            cps.append(pltpu.async_copy(
                xpad.at[idx_d.at[q]], rows_d.at[pl.ds(CH * q, CH)], sem_g))
        for cp in cps:
            cp.wait()
        row_base = blk * RB
        for q in range(10):
            for (j, r0, i0, L) in _RUNS[q]:
                pltpu.sync_copy(
                    rows_s.at[pl.ds(CH * q + i0, L)],
                    xs_out.at[pl.ds(row_base + r0, L), pl.ds(16 * j, 16)])
                pltpu.sync_copy(
                    rows_d.at[pl.ds(CH * q + i0, L)],
                    xd_out.at[pl.ds(row_base + r0, L), pl.ds(16 * j, 16)])
        return carry

    lax.fori_loop(0, nt, body, 0)


def _sc_gather(xpad, src, dst):
    f32 = jnp.float32
    mesh = plsc.VectorSubcoreMesh(
        core_axis_name="c", subcore_axis_name="s", num_cores=NC, num_subcores=NS)
    k = pl.kernel(
        _gather_body,
        out_type=[
            jax.ShapeDtypeStruct((E // 8, 128), f32),
            jax.ShapeDtypeStruct((E // 8, 128), f32),
        ],
        mesh=mesh,
        scratch_types=[
            pltpu.VMEM((10, CH), jnp.int32),
            pltpu.VMEM((10, CH), jnp.int32),
            pltpu.VMEM((10 * CH, 16), f32),
            pltpu.VMEM((10 * CH, 16), f32),
            pltpu.SemaphoreType.DMA,
            pltpu.SemaphoreType.DMA,
        ],
        compiler_params=pltpu.CompilerParams(use_tc_tiling_on_sc=False),
    )
    return k(xpad, src, dst)


def _mlp_body(xd_ref, xs_ref, w1d_ref, w1s_ref, b1_ref, w2_ref, b2_ref,
              w3g_ref, w3l_ref, b3g_ref, b3l_ref, grad_ref, msg_ref):
    f32 = jnp.float32
    xdp = xd_ref[...]
    xsp = xs_ref[...]
    xd = jnp.concatenate(
        [xdp[:, 16 * j:16 * (j + 1)] for j in range(8)], axis=0)
    xs = jnp.concatenate(
        [xsp[:, 16 * j:16 * (j + 1)] for j in range(8)], axis=0)
    h = jnp.dot(xd, w1d_ref[...], preferred_element_type=f32)
    h = h + jnp.dot(xs, w1s_ref[...], preferred_element_type=f32)
    h = _selu(h + b1_ref[...])
    h = _selu(jnp.dot(h, w2_ref[...], preferred_element_type=f32) + b2_ref[...])
    ag = jnp.dot(h, w3g_ref[...], preferred_element_type=f32) + b3g_ref[...]
    al = jnp.dot(h, w3l_ref[...], preferred_element_type=f32) + b3l_ref[...]
    d16 = xd - xs
    g16 = ag * d16
    m16 = al * d16
    grad_ref[...] = jnp.concatenate([g16[:, 0:F], g16[:, 8:8 + F]], axis=1)
    msg_ref[...] = jnp.concatenate(
        [m16[RB * j:RB * (j + 1), :] for j in range(8)], axis=1)


def _tc_mlp(xd_pk, xs_pk, weights):
    f32 = jnp.float32
    full = lambda shape: pl.BlockSpec(shape, lambda i: (0, 0))
    return pl.pallas_call(
        _mlp_body,
        grid=(E // EB,),
        in_specs=[
            pl.BlockSpec((RB, 128), lambda i: (i, 0)),
            pl.BlockSpec((RB, 128), lambda i: (i, 0)),
            full((16, 4 * H)),
            full((16, 4 * H)),
            full((1, 4 * H)),
            full((4 * H, 4 * H)),
            full((1, 4 * H)),
            full((4 * H, 16)),
            full((4 * H, 16)),
            full((1, 16)),
            full((1, 16)),
        ],
        out_specs=[
            pl.BlockSpec((EB, 2 * F), lambda i: (i, 0)),
            pl.BlockSpec((RB, 128), lambda i: (i, 0)),
        ],
        out_shape=[
            jax.ShapeDtypeStruct((E + N, 2 * F), f32),
            jax.ShapeDtypeStruct((E // 8, 128), f32),
        ],
        compiler_params=pltpu.CompilerParams(
            dimension_semantics=("arbitrary",)),
    )(xd_pk, xs_pk, *weights)


def _scatter_body(msg, dst, part_out, idx_d, rows_v, zbuf, acc, sem_i):
    c = lax.axis_index("c")
    s = lax.axis_index("s")
    wid = s * NC + c

    def zrow(i, carry):
        zbuf[i] = jnp.zeros((16,), jnp.float32)
        return carry

    lax.fori_loop(0, ZROWS, zrow, 0)
    for j in range(ROWS_PER_TILE // ZROWS):
        pltpu.sync_copy(zbuf, acc.at[pl.ds(s * ROWS_PER_TILE + j * ZROWS, ZROWS)])
    plsc.subcore_barrier()

    nt = lax.select(wid < NW - 1, CPW // 10, (NCHUNK - (NW - 1) * CPW) // 10)

    def body(u, carry):
        blk = wid * (CPW // 10) + u
        base = blk * EB
        row_base = blk * RB
        cps = []
        for q in range(10):
            cps.append(pltpu.async_copy(
                dst.at[pl.ds(base + CH * q, CH)], idx_d.at[q], sem_i))
            for (j, r0, i0, L) in _RUNS[q]:
                cps.append(pltpu.async_copy(
                    msg.at[pl.ds(row_base + r0, L), pl.ds(16 * j, 16)],
                    rows_v.at[pl.ds(CH * q + i0, L)], sem_i))
        for cp in cps:
            cp.wait()
        for q in range(10):
            pltpu.sync_copy(rows_v.at[pl.ds(CH * q, CH)],
                            acc.at[idx_d.at[q]], add=True)
        return carry

    lax.fori_loop(0, nt, body, 0)
    plsc.subcore_barrier()
    pltpu.sync_copy(acc.at[pl.ds(s * ROWS_PER_TILE, ROWS_PER_TILE)],
                    part_out.at[c, pl.ds(s * ROWS_PER_TILE, ROWS_PER_TILE)])


def _sc_scatter(msg_pk, dst):
    f32 = jnp.float32
    mesh = plsc.VectorSubcoreMesh(
        core_axis_name="c", subcore_axis_name="s", num_cores=NC, num_subcores=NS)
    k = pl.kernel(
        _scatter_body,
        out_type=jax.ShapeDtypeStruct((NC, N, 16), f32),
        mesh=mesh,
        scratch_types=[
            pltpu.VMEM((10, CH), jnp.int32),
            pltpu.VMEM((10 * CH, 16), f32),
            pltpu.VMEM((ZROWS, 16), f32),
            pltpu.VMEM_SHARED((N, 16), f32),
            pltpu.SemaphoreType.DMA,
        ],
        compiler_params=pltpu.CompilerParams(use_tc_tiling_on_sc=False),
    )
    return k(msg_pk, dst)


def _combine_body(buf_ref, part_ref, out_ref):
    del buf_ref
    p16 = part_ref[0] + part_ref[1]
    out_ref[...] = jnp.concatenate([p16[:, 0:F], p16[:, 8:8 + F]], axis=1)


def _tc_combine(buf, part):
    return pl.pallas_call(
        _combine_body,
        grid=(N // NB,),
        in_specs=[
            pl.BlockSpec((NB, 2 * F), lambda i: (E // NB + i, 0)),
            pl.BlockSpec((NC, NB, 16), lambda i: (0, i, 0)),
        ],
        out_specs=pl.BlockSpec((NB, 2 * F), lambda i: (E // NB + i, 0)),
        out_shape=jax.ShapeDtypeStruct((E + N, 2 * F), jnp.float32),
        input_output_aliases={0: 0},
        compiler_params=pltpu.CompilerParams(
            dimension_semantics=("arbitrary",)),
    )(buf, part)


def kernel(x, edge_index, GW1, Gb1, GW2, Gb2, GW3, Gb3,
           LW1, Lb1, LW2, Lb2, LW3, Lb3):
    src = edge_index[0]
    dst = edge_index[1]

    xpad = _tc_pack(x)
    xs_pk, xd_pk = _sc_gather(xpad, src, dst)
    weights = _prep_weights(GW1, Gb1, GW2, Gb2, GW3, Gb3,
                            LW1, Lb1, LW2, Lb2, LW3, Lb3)
    out_buf, msg_pk = _tc_mlp(xd_pk, xs_pk, weights)
    part = _sc_scatter(msg_pk, dst)
    return _tc_combine(out_buf, part)


# speedup vs baseline: 7.9202x; 1.2390x over previous
import functools

import jax
import jax.numpy as jnp
from jax import lax
from jax.experimental import pallas as pl
from jax.experimental.pallas import tpu as pltpu
from jax.experimental.pallas import tpu_sc as plsc

N = 50000
E = 800000
F = 5
H = 64
K = 2

NC = 2
NS = 16
NW = NC * NS

CH = 128
NCHUNK = E // CH
CPW = 200
ROWS_PER_TILE = N // NS
ZROWS = 625

EB = 3200
RB = EB // 8
CPB = EB // CH
NB = 2000
XB = 2000

_SELU_SCALE = 1.0507009873554805
_SELU_ALPHA = 1.6732632423543772

_RUNS = []
for _q in range(CPB):
    _k0 = CH * _q
    _r, _k, _i = [], _k0, 0
    while _k < _k0 + CH:
        _j, _rr = _k // RB, _k % RB
        _L = min(_k0 + CH, (_j + 1) * RB) - _k
        _r.append((_j, _rr, _i, _L))
        _k += _L
        _i += _L
    _RUNS.append(_r)


def _selu_unscaled(h):
    return jnp.where(h > 0, h, _SELU_ALPHA * (jnp.exp(h) - 1.0))


def _prep_weights(GW1, Gb1, GW2, Gb2, GW3, Gb3, LW1, Lb1, LW2, Lb2, LW3, Lb3):
    f32 = jnp.float32
    W1 = jnp.concatenate([GW1[0], GW1[1], LW1[0], LW1[1]], axis=1)
    W1d = jnp.zeros((16, 4 * H), f32).at[0:F, :].set(W1[0:F, :])
    W1s = jnp.zeros((16, 4 * H), f32).at[0:F, :].set(W1[F:2 * F, :])
    b1 = jnp.concatenate([Gb1[0], Gb1[1], Lb1[0], Lb1[1]])[None, :]

    W2 = jnp.zeros((4 * H, 4 * H), f32)
    for j, w in enumerate([GW2[0], GW2[1], LW2[0], LW2[1]]):
        W2 = W2.at[j * H:(j + 1) * H, j * H:(j + 1) * H].set(w)
    W2 = W2 * _SELU_SCALE
    b2 = jnp.concatenate([Gb2[0], Gb2[1], Lb2[0], Lb2[1]])[None, :]

    W3g = jnp.zeros((4 * H, 16), f32)
    W3g = W3g.at[0:H, 0:F].set(GW3[0][:, 0::2])
    W3g = W3g.at[H:2 * H, 8:8 + F].set(GW3[1][:, 0::2])
    W3l = jnp.zeros((4 * H, 16), f32)
    W3l = W3l.at[2 * H:3 * H, 0:F].set(LW3[0][:, 0::2])
    W3l = W3l.at[3 * H:4 * H, 8:8 + F].set(LW3[1][:, 0::2])
    W3g = W3g * _SELU_SCALE
    W3l = W3l * _SELU_SCALE
    b3g = jnp.zeros((1, 16), f32)
    b3g = b3g.at[0, 0:F].set(Gb3[0][0::2]).at[0, 8:8 + F].set(Gb3[1][0::2])
    b3l = jnp.zeros((1, 16), f32)
    b3l = b3l.at[0, 0:F].set(Lb3[0][0::2]).at[0, 8:8 + F].set(Lb3[1][0::2])
    return W1d, W1s, b1, W2, b2, W3g, W3l, b3g, b3l


def _pack_body(x_ref, out_ref):
    x = x_ref[...]
    z = jnp.zeros((XB, 3), jnp.float32)
    out_ref[...] = jnp.concatenate([x, z, x, z], axis=1)


def _tc_pack(x):
    return pl.pallas_call(
        _pack_body,
        grid=(N // XB,),
        in_specs=[pl.BlockSpec((XB, F), lambda i: (i, 0))],
        out_specs=pl.BlockSpec((XB, 16), lambda i: (i, 0)),
        out_shape=jax.ShapeDtypeStruct((N, 16), jnp.float32),
        compiler_params=pltpu.CompilerParams(
            dimension_semantics=("arbitrary",)),
    )(x)


def _gather_body(xpad, src, dst, xs_out, xd_out,
                 idx_s, idx_d, rows_s, rows_d, sem_i, sem_g):
    wid = lax.axis_index("s") * NC + lax.axis_index("c")
    g0 = wid * CPW
    nt = lax.select(wid < NW - 1, CPW // CPB, (NCHUNK - (NW - 1) * CPW) // CPB)

    def body(u, carry):
        blk = wid * (CPW // CPB) + u
        base = blk * EB
        cps = []
        for q in range(CPB):
            cps.append(pltpu.async_copy(
                src.at[pl.ds(base + CH * q, CH)], idx_s.at[q], sem_i))
            cps.append(pltpu.async_copy(
                dst.at[pl.ds(base + CH * q, CH)], idx_d.at[q], sem_i))
        for cp in cps:
            cp.wait()
        cps = []
        for q in range(CPB):
            cps.append(pltpu.async_copy(
                xpad.at[idx_s.at[q]], rows_s.at[pl.ds(CH * q, CH)], sem_g))
            cps.append(pltpu.async_copy(
                xpad.at[idx_d.at[q]], rows_d.at[pl.ds(CH * q, CH)], sem_g))
        for cp in cps:
            cp.wait()
        row_base = blk * RB
        for q in range(CPB):
            for (j, r0, i0, L) in _RUNS[q]:
                pltpu.sync_copy(
                    rows_s.at[pl.ds(CH * q + i0, L)],
                    xs_out.at[pl.ds(row_base + r0, L), pl.ds(16 * j, 16)])
                pltpu.sync_copy(
                    rows_d.at[pl.ds(CH * q + i0, L)],
                    xd_out.at[pl.ds(row_base + r0, L), pl.ds(16 * j, 16)])
        return carry

    lax.fori_loop(0, nt, body, 0)


def _sc_gather(xpad, src, dst):
    f32 = jnp.float32
    mesh = plsc.VectorSubcoreMesh(
        core_axis_name="c", subcore_axis_name="s", num_cores=NC, num_subcores=NS)
    k = pl.kernel(
        _gather_body,
        out_type=[
            jax.ShapeDtypeStruct((E // 8, 128), f32),
            jax.ShapeDtypeStruct((E // 8, 128), f32),
        ],
        mesh=mesh,
        scratch_types=[
            pltpu.VMEM((CPB, CH), jnp.int32),
            pltpu.VMEM((CPB, CH), jnp.int32),
            pltpu.VMEM((CPB * CH, 16), f32),
            pltpu.VMEM((CPB * CH, 16), f32),
            pltpu.SemaphoreType.DMA,
            pltpu.SemaphoreType.DMA,
        ],
        compiler_params=pltpu.CompilerParams(use_tc_tiling_on_sc=False),
    )
    return k(xpad, src, dst)


def _mlp_body(xd_ref, xs_ref, w1d_ref, w1s_ref, b1_ref, w2_ref, b2_ref,
              w3g_ref, w3l_ref, b3g_ref, b3l_ref, grad_ref, msg_ref):
    f32 = jnp.float32
    xdp = xd_ref[...]
    xsp = xs_ref[...]
    w1d = w1d_ref[...]
    w1s = w1s_ref[...]
    hs = []
    ds = []
    for j in range(8):
        xdj = xdp[:, 16 * j:16 * (j + 1)]
        xsj = xsp[:, 16 * j:16 * (j + 1)]
        hj = jnp.dot(xdj, w1d, preferred_element_type=f32)
        hj = hj + jnp.dot(xsj, w1s, preferred_element_type=f32)
        hs.append(hj)
        ds.append(xdj - xsj)
    h = jnp.concatenate(hs, axis=0)
    h = _selu_unscaled(h + b1_ref[...])
    h = _selu_unscaled(
        jnp.dot(h, w2_ref[...], preferred_element_type=f32) + b2_ref[...])
    ag = jnp.dot(h, w3g_ref[...], preferred_element_type=f32) + b3g_ref[...]
    al = jnp.dot(h, w3l_ref[...], preferred_element_type=f32) + b3l_ref[...]
    d16 = jnp.concatenate(ds, axis=0)
    g16 = ag * d16
    m16 = al * d16
    grad_ref[...] = jnp.concatenate([g16[:, 0:F], g16[:, 8:8 + F]], axis=1)
    msg_ref[...] = jnp.concatenate(
        [m16[RB * j:RB * (j + 1), :] for j in range(8)], axis=1)


def _tc_mlp(xd_pk, xs_pk, weights):
    f32 = jnp.float32
    full = lambda shape: pl.BlockSpec(shape, lambda i: (0, 0))
    return pl.pallas_call(
        _mlp_body,
        grid=(E // EB,),
        in_specs=[
            pl.BlockSpec((RB, 128), lambda i: (i, 0)),
            pl.BlockSpec((RB, 128), lambda i: (i, 0)),
            full((16, 4 * H)),
            full((16, 4 * H)),
            full((1, 4 * H)),
            full((4 * H, 4 * H)),
            full((1, 4 * H)),
            full((4 * H, 16)),
            full((4 * H, 16)),
            full((1, 16)),
            full((1, 16)),
        ],
        out_specs=[
            pl.BlockSpec((EB, 2 * F), lambda i: (i, 0)),
            pl.BlockSpec((RB, 128), lambda i: (i, 0)),
        ],
        out_shape=[
            jax.ShapeDtypeStruct((E + N, 2 * F), f32),
            jax.ShapeDtypeStruct((E // 8, 128), f32),
        ],
        compiler_params=pltpu.CompilerParams(
            dimension_semantics=("arbitrary",)),
    )(xd_pk, xs_pk, *weights)


def _scatter_body(msg, dst, part_out, idx_d, rows_v, zbuf, acc, sem_i):
    c = lax.axis_index("c")
    s = lax.axis_index("s")
    wid = s * NC + c

    def zrow(i, carry):
        zbuf[i] = jnp.zeros((16,), jnp.float32)
        return carry

    lax.fori_loop(0, ZROWS, zrow, 0)
    for j in range(ROWS_PER_TILE // ZROWS):
        pltpu.sync_copy(zbuf, acc.at[pl.ds(s * ROWS_PER_TILE + j * ZROWS, ZROWS)])
    plsc.subcore_barrier()

    nt = lax.select(wid < NW - 1, CPW // CPB, (NCHUNK - (NW - 1) * CPW) // CPB)

    def body(u, carry):
        blk = wid * (CPW // CPB) + u
        base = blk * EB
        row_base = blk * RB
        cps = []
        for q in range(CPB):
            cps.append(pltpu.async_copy(
                dst.at[pl.ds(base + CH * q, CH)], idx_d.at[q], sem_i))
            for (j, r0, i0, L) in _RUNS[q]:
                cps.append(pltpu.async_copy(
                    msg.at[pl.ds(row_base + r0, L), pl.ds(16 * j, 16)],
                    rows_v.at[pl.ds(CH * q + i0, L)], sem_i))
        for cp in cps:
            cp.wait()
        for q in range(CPB):
            pltpu.sync_copy(rows_v.at[pl.ds(CH * q, CH)],
                            acc.at[idx_d.at[q]], add=True)
        return carry

    lax.fori_loop(0, nt, body, 0)
    plsc.subcore_barrier()
    pltpu.sync_copy(acc.at[pl.ds(s * ROWS_PER_TILE, ROWS_PER_TILE)],
                    part_out.at[c, pl.ds(s * ROWS_PER_TILE, ROWS_PER_TILE)])


def _sc_scatter(msg_pk, dst):
    f32 = jnp.float32
    mesh = plsc.VectorSubcoreMesh(
        core_axis_name="c", subcore_axis_name="s", num_cores=NC, num_subcores=NS)
    k = pl.kernel(
        _scatter_body,
        out_type=jax.ShapeDtypeStruct((NC, N, 16), f32),
        mesh=mesh,
        scratch_types=[
            pltpu.VMEM((CPB, CH), jnp.int32),
            pltpu.VMEM((CPB * CH, 16), f32),
            pltpu.VMEM((ZROWS, 16), f32),
            pltpu.VMEM_SHARED((N, 16), f32),
            pltpu.SemaphoreType.DMA,
        ],
        compiler_params=pltpu.CompilerParams(use_tc_tiling_on_sc=False),
    )
    return k(msg_pk, dst)


def _combine_body(part_ref, out_ref):
    p16 = part_ref[0] + part_ref[1]
    out_ref[...] = jnp.concatenate([p16[:, 0:F], p16[:, 8:8 + F]], axis=1)


def _tc_combine(part):
    return pl.pallas_call(
        _combine_body,
        grid=(N // NB,),
        in_specs=[pl.BlockSpec((NC, NB, 16), lambda i: (0, i, 0))],
        out_specs=pl.BlockSpec((NB, 2 * F), lambda i: (i, 0)),
        out_shape=jax.ShapeDtypeStruct((N, 2 * F), jnp.float32),
        compiler_params=pltpu.CompilerParams(
            dimension_semantics=("arbitrary",)),
    )(part)


def kernel(x, edge_index, GW1, Gb1, GW2, Gb2, GW3, Gb3,
           LW1, Lb1, LW2, Lb2, LW3, Lb3):
    src = edge_index[0]
    dst = edge_index[1]

    xpad = _tc_pack(x)
    xs_pk, xd_pk = _sc_gather(xpad, src, dst)
    weights = _prep_weights(GW1, Gb1, GW2, Gb2, GW3, Gb3,
                            LW1, Lb1, LW2, Lb2, LW3, Lb3)
    out_buf, msg_pk = _tc_mlp(xd_pk, xs_pk, weights)
    part = _sc_scatter(msg_pk, dst)
    lap = _tc_combine(part)
    return lax.dynamic_update_slice(out_buf, lap, (E, 0))
